# baseline (speedup 1.0000x reference)
import functools

import jax
import jax.numpy as jnp
from jax.experimental import pallas as pl
from jax.experimental.pallas import tpu as pltpu

KHOPS = 8


def _pick_blk(n: int) -> int:
    for b in (400, 200, 100, 16, 8):
        if n % b == 0:
            return b
    return n


def _encoder_body(x_ref, w1_ref, b1_ref, w2_ref, b2_ref, h_ref, ht_ref):
    h1 = jnp.dot(x_ref[:], w1_ref[:], preferred_element_type=jnp.float32)
    h1 = jnp.maximum(h1 + b1_ref[:], 0.0)
    h = jnp.dot(h1, w2_ref[:], preferred_element_type=jnp.float32) + b2_ref[:]
    h_ref[:] = h
    ht_ref[:] = h.T


def _hop1_body(l_ref, h_ref, lbf_ref, t1_ref):
    lb = l_ref[:].astype(jnp.bfloat16)
    lbf_ref[:] = lb
    t1_ref[:] = jnp.dot(lb, h_ref[:].astype(jnp.bfloat16),
                        preferred_element_type=jnp.float32)


def _tr_body(t1_ref, t1t_ref):
    t1t_ref[:] = t1_ref[:].T


def _hops_body(l_ref, ht_ref, t1t_ref, g_ref, zo_ref, tbf_s, cache_s,
               *, blk, nc):
    hop = pl.program_id(0)
    i = pl.program_id(1)

    @pl.when(jnp.logical_and(hop == 0, i == 0))
    def _init():
        hh = ht_ref[:].T
        t1 = t1t_ref[:].T
        tbf_s[0] = hh.astype(jnp.bfloat16)
        tbf_s[1] = t1.astype(jnp.bfloat16)
        zo_ref[:] = g_ref[0:1, :] * hh + g_ref[1:2, :] * t1

    @pl.when(jnp.logical_and(hop == 0, i < nc))
    def _fill_cache():
        cache_s[pl.ds(i * blk, blk), :] = l_ref[:]

    ip = hop % 3
    ic = (hop + 1) % 3
    it = (hop + 2) % 3
    rows = pl.ds(i * blk, blk)
    tc = tbf_s[ic]
    acc = jax.lax.cond(
        jnp.logical_and(hop > 0, i < nc),
        lambda: jnp.dot(cache_s[pl.ds(i * blk, blk), :], tc,
                        preferred_element_type=jnp.float32),
        lambda: jnp.dot(l_ref[:], tc, preferred_element_type=jnp.float32),
    )
    tn = 2.0 * acc - tbf_s[ip, rows, :].astype(jnp.float32)
    tbf_s[it, rows, :] = tn.astype(jnp.bfloat16)
    gk = g_ref[pl.ds(hop + 2, 1), :]
    zo_ref[rows, :] += gk * tn


def kernel(x, L_tilde, W1, b1, W2, b2, gamma):
    n, in_dim = x.shape
    hid = W1.shape[1]
    f = W2.shape[1]
    blk = _pick_blk(n)
    nblk = n // blk

    g = jnp.broadcast_to(gamma[:, None], (KHOPS + 1, f)).astype(jnp.float32)
    b1r = b1.reshape(1, hid)
    b2r = b2.reshape(1, f)

    h, ht = pl.pallas_call(
        _encoder_body,
        out_shape=[
            jax.ShapeDtypeStruct((n, f), jnp.float32),
            jax.ShapeDtypeStruct((f, n), jnp.float32),
        ],
    )(x, W1, b1r, W2, b2r)

    lbf, t1 = pl.pallas_call(
        _hop1_body,
        grid=(nblk,),
        in_specs=[
            pl.BlockSpec((blk, n), lambda i: (i, 0)),
            pl.BlockSpec((n, f), lambda i: (0, 0)),
        ],
        out_specs=[
            pl.BlockSpec((blk, n), lambda i: (i, 0)),
            pl.BlockSpec((blk, f), lambda i: (i, 0)),
        ],
        out_shape=[
            jax.ShapeDtypeStruct((n, n), jnp.bfloat16),
            jax.ShapeDtypeStruct((n, f), jnp.float32),
        ],
        compiler_params=pltpu.CompilerParams(
            dimension_semantics=("parallel",)),
    )(L_tilde, h)

    t1t = pl.pallas_call(
        _tr_body,
        out_shape=jax.ShapeDtypeStruct((f, n), jnp.float32),
    )(t1)

    nc = 0
    full_tt = pl.BlockSpec((f, n), lambda h_, i: (0, 0))
    z = pl.pallas_call(
        functools.partial(_hops_body, blk=blk, nc=nc),
        grid=(KHOPS - 1, nblk),
        in_specs=[
            pl.BlockSpec(
                (blk, n),
                lambda h_, i: (jnp.where(h_ == 0, i, jnp.maximum(i, nc)), 0)),
            full_tt,
            full_tt,
            pl.BlockSpec((KHOPS + 1, f), lambda h_, i: (0, 0)),
        ],
        out_specs=pl.BlockSpec((n, f), lambda h_, i: (0, 0)),
        out_shape=jax.ShapeDtypeStruct((n, f), jnp.float32),
        scratch_shapes=[
            pltpu.VMEM((3, n, f), jnp.bfloat16),
            pltpu.VMEM((max(nc, 1) * blk, n), jnp.bfloat16),
        ],
        compiler_params=pltpu.CompilerParams(
            dimension_semantics=("arbitrary", "arbitrary"),
            vmem_limit_bytes=128 * 1024 * 1024),
    )(lbf, ht, t1t, g)
    return z


# speedup vs baseline: 1.1079x; 1.1079x over previous
import functools

import jax
import jax.numpy as jnp
from jax.experimental import pallas as pl
from jax.experimental.pallas import tpu as pltpu

KHOPS = 8


def _pick_blk(n: int) -> int:
    for b in (400, 200, 100, 16, 8):
        if n % b == 0:
            return b
    return n


def _encoder_body(x_ref, w1_ref, b1_ref, w2_ref, b2_ref, h_ref):
    h1 = jnp.dot(x_ref[:], w1_ref[:], preferred_element_type=jnp.float32)
    h1 = jnp.maximum(h1 + b1_ref[:], 0.0)
    h_ref[:] = jnp.dot(h1, w2_ref[:],
                       preferred_element_type=jnp.float32) + b2_ref[:]


def _hop1_body(l_ref, h_ref, lbf_ref, t1_ref):
    lb = l_ref[:].astype(jnp.bfloat16)
    lbf_ref[:] = lb
    t1_ref[:] = jnp.dot(lb, h_ref[:].astype(jnp.bfloat16),
                        preferred_element_type=jnp.float32)


def _hops_body(l_ref, h_ref, t1_ref, g_ref, zo_ref, tbf_s, *, blk, n):
    hop = pl.program_id(0)
    i = pl.program_id(1)

    @pl.when(jnp.logical_and(hop == 0, i == 0))
    def _init():
        hh = h_ref[:]
        t1 = t1_ref[:]
        tbf_s[0, 0:n, :] = hh.astype(jnp.bfloat16)
        tbf_s[1, 0:n, :] = t1.astype(jnp.bfloat16)
        zo_ref[0:n, :] = g_ref[0:1, :] * hh + g_ref[1:2, :] * t1

    ip = hop % 3
    ic = (hop + 1) % 3
    it = (hop + 2) % 3
    rows = pl.ds(i * blk, blk)
    tc = tbf_s[ic, 0:n, :]
    acc = jnp.dot(l_ref[:], tc, preferred_element_type=jnp.float32)
    tn = 2.0 * acc - tbf_s[ip, rows, :].astype(jnp.float32)
    tbf_s[it, rows, :] = tn.astype(jnp.bfloat16)
    gk = g_ref[pl.ds(hop + 2, 1), :]
    zo_ref[rows, :] += gk * tn


def kernel(x, L_tilde, W1, b1, W2, b2, gamma):
    n, in_dim = x.shape
    hid = W1.shape[1]
    f = W2.shape[1]
    blk = _pick_blk(n)
    nblk = n // blk

    g = jnp.broadcast_to(gamma[:, None], (KHOPS + 1, f)).astype(jnp.float32)
    b1r = b1.reshape(1, hid)
    b2r = b2.reshape(1, f)

    h = pl.pallas_call(
        _encoder_body,
        out_shape=jax.ShapeDtypeStruct((n, f), jnp.float32),
    )(x, W1, b1r, W2, b2r)

    lbf, t1 = pl.pallas_call(
        _hop1_body,
        grid=(nblk,),
        in_specs=[
            pl.BlockSpec((blk, n), lambda i: (i, 0)),
            pl.BlockSpec((n, f), lambda i: (0, 0)),
        ],
        out_specs=[
            pl.BlockSpec((blk, n), lambda i: (i, 0)),
            pl.BlockSpec((blk, f), lambda i: (i, 0)),
        ],
        out_shape=[
            jax.ShapeDtypeStruct((n, n), jnp.bfloat16),
            jax.ShapeDtypeStruct((n, f), jnp.float32),
        ],
        compiler_params=pltpu.CompilerParams(
            dimension_semantics=("parallel",)),
    )(L_tilde, h)

    blkf = 800 if n % 400 == 0 else blk
    nblkf = pl.cdiv(n, blkf)
    npad = nblkf * blkf
    full_t = pl.BlockSpec((n, f), lambda h_, i: (0, 0))
    zp = pl.pallas_call(
        functools.partial(_hops_body, blk=blkf, n=n),
        grid=(KHOPS - 1, nblkf),
        in_specs=[
            pl.BlockSpec((blkf, n), lambda h_, i: (i, 0)),
            full_t,
            full_t,
            pl.BlockSpec((KHOPS + 1, f), lambda h_, i: (0, 0)),
        ],
        out_specs=pl.BlockSpec((npad, f), lambda h_, i: (0, 0)),
        out_shape=jax.ShapeDtypeStruct((npad, f), jnp.float32),
        scratch_shapes=[
            pltpu.VMEM((3, npad, f), jnp.bfloat16),
        ],
        compiler_params=pltpu.CompilerParams(
            dimension_semantics=("arbitrary", "arbitrary"),
            vmem_limit_bytes=128 * 1024 * 1024),
    )(lbf, h, t1, g)
    return zp[:n]


# speedup vs baseline: 1.1188x; 1.0098x over previous
import functools

import jax
import jax.numpy as jnp
from jax.experimental import pallas as pl
from jax.experimental.pallas import tpu as pltpu

KHOPS = 8


def _pick_blk(n: int) -> int:
    for b in (400, 200, 100, 16, 8):
        if n % b == 0:
            return b
    return n


def _encoder_body(x_ref, w1_ref, b1_ref, w2_ref, b2_ref, h_ref):
    h1 = jnp.dot(x_ref[:], w1_ref[:], preferred_element_type=jnp.float32)
    h1 = jnp.maximum(h1 + b1_ref[:], 0.0)
    h_ref[:] = jnp.dot(h1, w2_ref[:],
                       preferred_element_type=jnp.float32) + b2_ref[:]


def _hop1_body(l_ref, h_ref, lbf_ref, ht1_ref):
    lb = l_ref[:].astype(jnp.bfloat16)
    lbf_ref[:] = lb
    t1 = jnp.dot(lb, h_ref[:].astype(jnp.bfloat16),
                 preferred_element_type=jnp.float32)
    i = pl.program_id(0)
    blk = l_ref.shape[0]
    ht1_ref[:] = jnp.concatenate(
        [h_ref[pl.ds(i * blk, blk), :], t1], axis=1)


def _hops_body(l_ref, ht1_ref, g_ref, zo_ref, tbf_s, *, blk, n, f):
    hop = pl.program_id(0)
    i = pl.program_id(1)

    @pl.when(jnp.logical_and(hop == 0, i == 0))
    def _init():
        ht1 = ht1_ref[:]
        hh = ht1[:, 0:f]
        t1 = ht1[:, f:2 * f]
        tbf_s[0, 0:n, :] = hh.astype(jnp.bfloat16)
        tbf_s[1, 0:n, :] = t1.astype(jnp.bfloat16)
        zo_ref[0:n, :] = g_ref[0:1, :] * hh + g_ref[1:2, :] * t1

    ip = hop % 3
    ic = (hop + 1) % 3
    it = (hop + 2) % 3
    rows = pl.ds(i * blk, blk)
    tc = tbf_s[ic, 0:n, :]
    acc = jnp.dot(l_ref[:], tc, preferred_element_type=jnp.float32)
    tn = 2.0 * acc - tbf_s[ip, rows, :].astype(jnp.float32)
    tbf_s[it, rows, :] = tn.astype(jnp.bfloat16)
    gk = g_ref[pl.ds(hop + 2, 1), :]
    zo_ref[rows, :] += gk * tn


def kernel(x, L_tilde, W1, b1, W2, b2, gamma):
    n, in_dim = x.shape
    hid = W1.shape[1]
    f = W2.shape[1]
    blk = _pick_blk(n)
    nblk = n // blk

    g = jnp.broadcast_to(gamma[:, None], (KHOPS + 1, f)).astype(jnp.float32)
    b1r = b1.reshape(1, hid)
    b2r = b2.reshape(1, f)

    h = pl.pallas_call(
        _encoder_body,
        out_shape=jax.ShapeDtypeStruct((n, f), jnp.float32),
    )(x, W1, b1r, W2, b2r)

    lbf, ht1 = pl.pallas_call(
        _hop1_body,
        grid=(nblk,),
        in_specs=[
            pl.BlockSpec((blk, n), lambda i: (i, 0)),
            pl.BlockSpec((n, f), lambda i: (0, 0)),
        ],
        out_specs=[
            pl.BlockSpec((blk, n), lambda i: (i, 0)),
            pl.BlockSpec((blk, 2 * f), lambda i: (i, 0)),
        ],
        out_shape=[
            jax.ShapeDtypeStruct((n, n), jnp.bfloat16),
            jax.ShapeDtypeStruct((n, 2 * f), jnp.float32),
        ],
        compiler_params=pltpu.CompilerParams(
            dimension_semantics=("parallel",)),
    )(L_tilde, h)

    blkf = 1040 if n % 400 == 0 else blk
    nblkf = pl.cdiv(n, blkf)
    npad = nblkf * blkf
    zp = pl.pallas_call(
        functools.partial(_hops_body, blk=blkf, n=n, f=f),
        grid=(KHOPS - 1, nblkf),
        in_specs=[
            pl.BlockSpec((blkf, n), lambda h_, i: (i, 0)),
            pl.BlockSpec((n, 2 * f), lambda h_, i: (0, 0)),
            pl.BlockSpec((KHOPS + 1, f), lambda h_, i: (0, 0)),
        ],
        out_specs=pl.BlockSpec((npad, f), lambda h_, i: (0, 0)),
        out_shape=jax.ShapeDtypeStruct((npad, f), jnp.float32),
        scratch_shapes=[
            pltpu.VMEM((3, npad, f), jnp.bfloat16),
        ],
        compiler_params=pltpu.CompilerParams(
            dimension_semantics=("arbitrary", "arbitrary"),
            vmem_limit_bytes=128 * 1024 * 1024),
    )(lbf, ht1, g)
    return zp[:n]


# speedup vs baseline: 1.1284x; 1.0086x over previous
import functools

import jax
import jax.numpy as jnp
from jax.experimental import pallas as pl
from jax.experimental.pallas import tpu as pltpu

KHOPS = 8


def _pick_blk(n: int) -> int:
    for b in (400, 200, 100, 16, 8):
        if n % b == 0:
            return b
    return n


def _hop1_body(x_ref, w1_ref, b1_ref, w2_ref, b2_ref, l_ref,
               lbf_ref, ht1_ref, h_s):
    i = pl.program_id(0)

    @pl.when(i == 0)
    def _encode():
        h1 = jnp.dot(x_ref[:], w1_ref[:], preferred_element_type=jnp.float32)
        h1 = jnp.maximum(h1 + b1_ref[:], 0.0)
        h_s[:] = jnp.dot(h1, w2_ref[:],
                         preferred_element_type=jnp.float32) + b2_ref[:]

    lb = l_ref[:].astype(jnp.bfloat16)
    lbf_ref[:] = lb
    t1 = jnp.dot(lb, h_s[:].astype(jnp.bfloat16),
                 preferred_element_type=jnp.float32)
    blk = l_ref.shape[0]
    ht1_ref[:] = jnp.concatenate(
        [h_s[pl.ds(i * blk, blk), :], t1], axis=1)


def _hops_body(l_ref, ht1_ref, g_ref, zo_ref, tbf_s, *, blk, n, f):
    hop = pl.program_id(0)
    i = pl.program_id(1)

    @pl.when(jnp.logical_and(hop == 0, i == 0))
    def _init():
        ht1 = ht1_ref[:]
        hh = ht1[:, 0:f]
        t1 = ht1[:, f:2 * f]
        tbf_s[0, 0:n, :] = hh.astype(jnp.bfloat16)
        tbf_s[1, 0:n, :] = t1.astype(jnp.bfloat16)
        zo_ref[0:n, :] = g_ref[0:1, :] * hh + g_ref[1:2, :] * t1

    ip = hop % 3
    ic = (hop + 1) % 3
    it = (hop + 2) % 3
    rows = pl.ds(i * blk, blk)
    tc = tbf_s[ic, 0:n, :]
    acc = jnp.dot(l_ref[:], tc, preferred_element_type=jnp.float32)
    tn = 2.0 * acc - tbf_s[ip, rows, :].astype(jnp.float32)
    tbf_s[it, rows, :] = tn.astype(jnp.bfloat16)
    gk = g_ref[pl.ds(hop + 2, 1), :]
    zo_ref[rows, :] += gk * tn


def kernel(x, L_tilde, W1, b1, W2, b2, gamma):
    n, in_dim = x.shape
    hid = W1.shape[1]
    f = W2.shape[1]
    blk = _pick_blk(n)
    nblk = n // blk

    g = jnp.broadcast_to(gamma[:, None], (KHOPS + 1, f)).astype(jnp.float32)
    b1r = b1.reshape(1, hid)
    b2r = b2.reshape(1, f)

    full = lambda a, b: pl.BlockSpec((a, b), lambda i: (0, 0))
    lbf, ht1 = pl.pallas_call(
        _hop1_body,
        grid=(nblk,),
        in_specs=[
            full(n, in_dim),
            full(in_dim, hid),
            full(1, hid),
            full(hid, f),
            full(1, f),
            pl.BlockSpec((blk, n), lambda i: (i, 0)),
        ],
        out_specs=[
            pl.BlockSpec((blk, n), lambda i: (i, 0)),
            pl.BlockSpec((blk, 2 * f), lambda i: (i, 0)),
        ],
        out_shape=[
            jax.ShapeDtypeStruct((n, n), jnp.bfloat16),
            jax.ShapeDtypeStruct((n, 2 * f), jnp.float32),
        ],
        scratch_shapes=[pltpu.VMEM((n, f), jnp.float32)],
        compiler_params=pltpu.CompilerParams(
            dimension_semantics=("arbitrary",),
            vmem_limit_bytes=128 * 1024 * 1024),
    )(x, W1, b1r, W2, b2r, L_tilde)

    blkf = 1040 if n % 400 == 0 else blk
    nblkf = pl.cdiv(n, blkf)
    npad = nblkf * blkf
    zp = pl.pallas_call(
        functools.partial(_hops_body, blk=blkf, n=n, f=f),
        grid=(KHOPS - 1, nblkf),
        in_specs=[
            pl.BlockSpec((blkf, n), lambda h_, i: (i, 0)),
            pl.BlockSpec((n, 2 * f), lambda h_, i: (0, 0)),
            pl.BlockSpec((KHOPS + 1, f), lambda h_, i: (0, 0)),
        ],
        out_specs=pl.BlockSpec((npad, f), lambda h_, i: (0, 0)),
        out_shape=jax.ShapeDtypeStruct((npad, f), jnp.float32),
        scratch_shapes=[
            pltpu.VMEM((3, npad, f), jnp.bfloat16),
        ],
        compiler_params=pltpu.CompilerParams(
            dimension_semantics=("arbitrary", "arbitrary"),
            vmem_limit_bytes=128 * 1024 * 1024),
    )(lbf, ht1, g)
    return zp[:n]


# speedup vs baseline: 1.1305x; 1.0018x over previous
import functools

import jax
import jax.numpy as jnp
from jax.experimental import pallas as pl
from jax.experimental.pallas import tpu as pltpu

KHOPS = 8


def _pick_blk(n: int) -> int:
    for b in (400, 200, 100, 16, 8):
        if n % b == 0:
            return b
    return n


def _hop1_body(x_ref, w1_ref, b1_ref, w2_ref, b2_ref, l_ref,
               lbf_ref, ht1_ref, h_s):
    i = pl.program_id(0)

    @pl.when(i == 0)
    def _encode():
        h1 = jnp.dot(x_ref[:], w1_ref[:], preferred_element_type=jnp.float32)
        h1 = jnp.maximum(h1 + b1_ref[:], 0.0)
        h_s[:] = jnp.dot(h1, w2_ref[:],
                         preferred_element_type=jnp.float32) + b2_ref[:]

    lb = l_ref[:].astype(jnp.bfloat16)
    lbf_ref[:] = lb
    t1 = jnp.dot(lb, h_s[:].astype(jnp.bfloat16),
                 preferred_element_type=jnp.float32)
    blk = l_ref.shape[0]
    ht1_ref[:] = jnp.concatenate(
        [h_s[pl.ds(i * blk, blk), :], t1], axis=1).astype(jnp.bfloat16)


def _hops_body(l_ref, ht1_ref, g_ref, zo_ref, tbf_s, *, blk, n, f):
    hop = pl.program_id(0)
    i = pl.program_id(1)

    @pl.when(jnp.logical_and(hop == 0, i == 0))
    def _init():
        ht1 = ht1_ref[:]
        hh = ht1[:, 0:f]
        t1 = ht1[:, f:2 * f]
        tbf_s[0, 0:n, :] = hh.astype(jnp.bfloat16)
        tbf_s[1, 0:n, :] = t1.astype(jnp.bfloat16)
        zo_ref[0:n, :] = g_ref[0:1, :] * hh + g_ref[1:2, :] * t1

    ip = hop % 3
    ic = (hop + 1) % 3
    it = (hop + 2) % 3
    rows = pl.ds(i * blk, blk)
    tc = tbf_s[ic, 0:n, :]
    acc = jnp.dot(l_ref[:], tc, preferred_element_type=jnp.float32)
    tn = 2.0 * acc - tbf_s[ip, rows, :].astype(jnp.float32)
    tbf_s[it, rows, :] = tn.astype(jnp.bfloat16)
    gk = g_ref[pl.ds(hop + 2, 1), :]
    zo_ref[rows, :] += gk * tn


def kernel(x, L_tilde, W1, b1, W2, b2, gamma):
    n, in_dim = x.shape
    hid = W1.shape[1]
    f = W2.shape[1]
    blk = _pick_blk(n)
    nblk = n // blk

    g = jnp.broadcast_to(gamma[:, None], (KHOPS + 1, f)).astype(jnp.float32)
    b1r = b1.reshape(1, hid)
    b2r = b2.reshape(1, f)

    full = lambda a, b: pl.BlockSpec((a, b), lambda i: (0, 0))
    lbf, ht1 = pl.pallas_call(
        _hop1_body,
        grid=(nblk,),
        in_specs=[
            full(n, in_dim),
            full(in_dim, hid),
            full(1, hid),
            full(hid, f),
            full(1, f),
            pl.BlockSpec((blk, n), lambda i: (i, 0)),
        ],
        out_specs=[
            pl.BlockSpec((blk, n), lambda i: (i, 0)),
            pl.BlockSpec((blk, 2 * f), lambda i: (i, 0)),
        ],
        out_shape=[
            jax.ShapeDtypeStruct((n, n), jnp.bfloat16),
            jax.ShapeDtypeStruct((n, 2 * f), jnp.bfloat16),
        ],
        scratch_shapes=[pltpu.VMEM((n, f), jnp.float32)],
        compiler_params=pltpu.CompilerParams(
            dimension_semantics=("arbitrary",),
            vmem_limit_bytes=128 * 1024 * 1024),
    )(x, W1, b1r, W2, b2r, L_tilde)

    blkf = 1040 if n % 400 == 0 else blk
    nblkf = pl.cdiv(n, blkf)
    npad = nblkf * blkf
    zp = pl.pallas_call(
        functools.partial(_hops_body, blk=blkf, n=n, f=f),
        grid=(KHOPS - 1, nblkf),
        in_specs=[
            pl.BlockSpec((blkf, n), lambda h_, i: (i, 0)),
            pl.BlockSpec((n, 2 * f), lambda h_, i: (0, 0)),
            pl.BlockSpec((KHOPS + 1, f), lambda h_, i: (0, 0)),
        ],
        out_specs=pl.BlockSpec((npad, f), lambda h_, i: (0, 0)),
        out_shape=jax.ShapeDtypeStruct((npad, f), jnp.float32),
        scratch_shapes=[
            pltpu.VMEM((3, npad, f), jnp.bfloat16),
        ],
        compiler_params=pltpu.CompilerParams(
            dimension_semantics=("arbitrary", "arbitrary"),
            vmem_limit_bytes=128 * 1024 * 1024),
    )(lbf, ht1, g)
    return zp[:n]
